# Initial kernel scaffold; baseline (speedup 1.0000x reference)
import jax
import jax.numpy as jnp
from jax.experimental import pallas as pl


def kernel(x, batch, W_ih, W_hh, b_ih, b_hh):
    raise NotImplementedError("write your pallas kernel here")


# speedup vs baseline: 4.2018x; 4.2018x over previous
import functools

import jax
import jax.numpy as jnp
from jax import lax
from jax.experimental import pallas as pl
from jax.experimental.pallas import tpu as pltpu

N = 100000
D = 128
B = 512
STEPS = 4
TILE = 2000
NT = N // TILE

_PREC = lax.Precision.HIGHEST
_NEG = -1e30


def _dot_nt(a, b):
    return lax.dot_general(a, b, (((1,), (1,)), ((), ())),
                           precision=_PREC, preferred_element_type=jnp.float32)


def _dot_nn(a, b):
    return lax.dot_general(a, b, (((1,), (0,)), ((), ())),
                           precision=_PREC, preferred_element_type=jnp.float32)


def _set2set_kernel(x_ref, b_ref, wih_ref, whh_ref, bias_ref, out_ref):
    wih = wih_ref[...]
    whh = whh_ref[...]
    bias = bias_ref[...]

    q = jnp.zeros((B, D), jnp.float32)
    h = jnp.zeros((B, D), jnp.float32)
    c = jnp.zeros((B, D), jnp.float32)
    r = jnp.zeros((B, D), jnp.float32)

    seg_iota = lax.broadcasted_iota(jnp.int32, (B, TILE), 0)

    for _ in range(STEPS):
        def tile_body(t, carry):
            m, s, S, cnt = carry
            xt = x_ref[t]
            btr = b_ref[t]
            oh = seg_iota == btr
            E = _dot_nt(q, xt)
            e_row = jnp.sum(jnp.where(oh, E, 0.0), axis=0, keepdims=True)
            mt = jnp.max(jnp.where(oh, E, _NEG), axis=1, keepdims=True)
            m_new = jnp.maximum(m, mt)
            scale = jnp.exp(m - m_new)
            g_row = jnp.sum(jnp.where(oh, m_new, 0.0), axis=0, keepdims=True)
            ex = jnp.exp(e_row - g_row)
            wT = jnp.where(oh, ex, 0.0)
            s_new = s * scale + jnp.sum(wT, axis=1, keepdims=True)
            S_new = S * scale + _dot_nn(wT, xt)
            cnt_new = cnt + jnp.sum(oh.astype(jnp.float32), axis=1, keepdims=True)
            return m_new, s_new, S_new, cnt_new

        init = (jnp.full((B, 1), _NEG, jnp.float32),
                jnp.zeros((B, 1), jnp.float32),
                jnp.zeros((B, D), jnp.float32),
                jnp.zeros((B, 1), jnp.float32))
        m, s, S, cnt = lax.fori_loop(0, NT, tile_body, init)

        r = jnp.where(s > 0.0, S / jnp.where(s > 0.0, s, 1.0), 0.0)
        r = r / jnp.maximum(cnt, 1.0)

        inp = jnp.concatenate([q, r], axis=1)
        gates = _dot_nt(inp, wih) + _dot_nt(h, whh) + bias
        i_g = jax.nn.sigmoid(gates[:, 0:D])
        f_g = jax.nn.sigmoid(gates[:, D:2 * D])
        g_g = jnp.tanh(gates[:, 2 * D:3 * D])
        o_g = jax.nn.sigmoid(gates[:, 3 * D:4 * D])
        c = f_g * c + i_g * g_g
        h = o_g * jnp.tanh(c)
        q = h

    out_ref[...] = jnp.concatenate([q, r], axis=1)


@jax.jit
def kernel(x, batch, W_ih, W_hh, b_ih, b_hh):
    x3 = x.reshape(NT, TILE, D)
    b3 = batch.astype(jnp.int32).reshape(NT, 1, TILE)
    bias = (b_ih + b_hh).reshape(1, 4 * D)
    return pl.pallas_call(
        _set2set_kernel,
        out_shape=jax.ShapeDtypeStruct((B, 2 * D), jnp.float32),
        compiler_params=pltpu.CompilerParams(
            vmem_limit_bytes=120 * 1024 * 1024,
        ),
    )(x3, b3, W_ih, W_hh, bias)


# speedup vs baseline: 11.6002x; 2.7608x over previous
import functools

import jax
import jax.numpy as jnp
from jax import lax
from jax.experimental import pallas as pl
from jax.experimental.pallas import tpu as pltpu

N = 100000
D = 128
B = 512
STEPS = 4
TILE = 2000
NT = N // TILE

_NEG = -1e30


def _dot_nt(a, b, precision=lax.Precision.HIGHEST):
    return lax.dot_general(a, b, (((1,), (1,)), ((), ())),
                           precision=precision, preferred_element_type=jnp.float32)


def _dot_nn(a, b, precision=lax.Precision.HIGHEST):
    return lax.dot_general(a, b, (((1,), (0,)), ((), ())),
                           precision=precision, preferred_element_type=jnp.float32)


def _set2set_kernel(x_ref, b_ref, wih_ref, whh_ref, bias_ref, out_ref):
    wih = wih_ref[...]
    whh = whh_ref[...]
    bias = bias_ref[...]

    q = jnp.zeros((B, D), jnp.float32)
    h = jnp.zeros((B, D), jnp.float32)
    c = jnp.zeros((B, D), jnp.float32)
    r = jnp.zeros((B, D), jnp.float32)

    seg_iota = lax.broadcasted_iota(jnp.int32, (B, TILE), 0)

    for _ in range(STEPS):
        def tile_body(t, carry):
            m, s, S, cnt = carry
            xt = x_ref[t]
            btr = b_ref[t]
            oh = seg_iota == btr
            E = _dot_nt(q, xt, lax.Precision.DEFAULT)
            e_row = jnp.sum(jnp.where(oh, E, 0.0), axis=0, keepdims=True)
            mt = jnp.max(jnp.where(oh, E, _NEG), axis=1, keepdims=True)
            m_new = jnp.maximum(m, mt)
            scale = jnp.exp(m - m_new)
            g_row = jnp.sum(jnp.where(oh, m_new, 0.0), axis=0, keepdims=True)
            ex = jnp.exp(e_row - g_row)
            wT = jnp.where(oh, ex, 0.0)
            s_new = s * scale + jnp.sum(wT, axis=1, keepdims=True)
            S_new = S * scale + _dot_nn(wT, xt, lax.Precision.DEFAULT)
            cnt_new = cnt + jnp.sum(oh.astype(jnp.float32), axis=1, keepdims=True)
            return m_new, s_new, S_new, cnt_new

        init = (jnp.full((B, 1), _NEG, jnp.float32),
                jnp.zeros((B, 1), jnp.float32),
                jnp.zeros((B, D), jnp.float32),
                jnp.zeros((B, 1), jnp.float32))
        m, s, S, cnt = lax.fori_loop(0, NT, tile_body, init)

        r = jnp.where(s > 0.0, S / jnp.where(s > 0.0, s, 1.0), 0.0)
        r = r / jnp.maximum(cnt, 1.0)

        inp = jnp.concatenate([q, r], axis=1)
        gates = _dot_nt(inp, wih) + _dot_nt(h, whh) + bias
        i_g = jax.nn.sigmoid(gates[:, 0:D])
        f_g = jax.nn.sigmoid(gates[:, D:2 * D])
        g_g = jnp.tanh(gates[:, 2 * D:3 * D])
        o_g = jax.nn.sigmoid(gates[:, 3 * D:4 * D])
        c = f_g * c + i_g * g_g
        h = o_g * jnp.tanh(c)
        q = h

    out_ref[...] = jnp.concatenate([q, r], axis=1)


@jax.jit
def kernel(x, batch, W_ih, W_hh, b_ih, b_hh):
    x3 = x.reshape(NT, TILE, D)
    b3 = batch.astype(jnp.int32).reshape(NT, 1, TILE)
    bias = (b_ih + b_hh).reshape(1, 4 * D)
    return pl.pallas_call(
        _set2set_kernel,
        out_shape=jax.ShapeDtypeStruct((B, 2 * D), jnp.float32),
        compiler_params=pltpu.CompilerParams(
            vmem_limit_bytes=120 * 1024 * 1024,
        ),
    )(x3, b3, W_ih, W_hh, bias)


# speedup vs baseline: 25.9831x; 2.2399x over previous
import jax
import jax.numpy as jnp
from jax import lax
from jax.experimental import pallas as pl
from jax.experimental.pallas import tpu as pltpu

N = 100000
D = 128
B = 512
STEPS = 4
TILE = 2000
NT = N // TILE
W = 128

_NEG = -1e30


def _dot_nt(a, b, precision=lax.Precision.HIGHEST):
    return lax.dot_general(a, b, (((1,), (1,)), ((), ())),
                           precision=precision, preferred_element_type=jnp.float32)


def _dot_nn(a, b, precision=lax.Precision.HIGHEST):
    return lax.dot_general(a, b, (((1,), (0,)), ((), ())),
                           precision=precision, preferred_element_type=jnp.float32)


def _set2set_kernel(x_ref, b_ref, wih_ref, whh_ref, bias_ref, out_ref,
                    q_ref, m_ref, s_ref, cnt_ref, S_ref):
    wih = wih_ref[...]
    whh = whh_ref[...]
    bias = bias_ref[...]

    q_ref[...] = jnp.zeros((B, D), jnp.float32)
    h = jnp.zeros((B, D), jnp.float32)
    c = jnp.zeros((B, D), jnp.float32)

    iota_w = lax.broadcasted_iota(jnp.int32, (W, TILE), 0)
    iota_f = lax.broadcasted_iota(jnp.int32, (B, TILE), 0)

    r = None
    for _ in range(STEPS):
        m_ref[...] = jnp.full((B, 1), _NEG, jnp.float32)
        s_ref[...] = jnp.zeros((B, 1), jnp.float32)
        cnt_ref[...] = jnp.zeros((B, 1), jnp.float32)
        S_ref[...] = jnp.zeros((B, D), jnp.float32)

        def tile_body(t, carry):
            xt = x_ref[t]
            btr = b_ref[t]
            b0a = jnp.minimum((jnp.min(btr) // 8) * 8, B - W)
            wide = (jnp.max(btr) - b0a) >= W

            @pl.when(jnp.logical_not(wide))
            def narrow():
                oh = (iota_w + b0a) == btr
                qw = q_ref[pl.ds(b0a, W), :]
                E = _dot_nt(qw, xt, lax.Precision.DEFAULT)
                e_row = jnp.sum(jnp.where(oh, E, 0.0), axis=0, keepdims=True)
                mt = jnp.max(jnp.where(oh, E, _NEG), axis=1, keepdims=True)
                mw = m_ref[pl.ds(b0a, W), :]
                mw_new = jnp.maximum(mw, mt)
                scale = jnp.exp(mw - mw_new)
                g_row = jnp.sum(jnp.where(oh, mw_new, 0.0), axis=0, keepdims=True)
                ex = jnp.exp(e_row - g_row)
                wT = jnp.where(oh, ex, 0.0)
                m_ref[pl.ds(b0a, W), :] = mw_new
                s_ref[pl.ds(b0a, W), :] = (s_ref[pl.ds(b0a, W), :] * scale
                                           + jnp.sum(wT, axis=1, keepdims=True))
                S_ref[pl.ds(b0a, W), :] = (S_ref[pl.ds(b0a, W), :] * scale
                                           + _dot_nn(wT, xt, lax.Precision.DEFAULT))
                cnt_ref[pl.ds(b0a, W), :] = (cnt_ref[pl.ds(b0a, W), :]
                                             + jnp.sum(oh.astype(jnp.float32),
                                                       axis=1, keepdims=True))

            @pl.when(wide)
            def full():
                oh = iota_f == btr
                E = _dot_nt(q_ref[...], xt, lax.Precision.DEFAULT)
                e_row = jnp.sum(jnp.where(oh, E, 0.0), axis=0, keepdims=True)
                mt = jnp.max(jnp.where(oh, E, _NEG), axis=1, keepdims=True)
                m = m_ref[...]
                m_new = jnp.maximum(m, mt)
                scale = jnp.exp(m - m_new)
                g_row = jnp.sum(jnp.where(oh, m_new, 0.0), axis=0, keepdims=True)
                ex = jnp.exp(e_row - g_row)
                wT = jnp.where(oh, ex, 0.0)
                m_ref[...] = m_new
                s_ref[...] = s_ref[...] * scale + jnp.sum(wT, axis=1, keepdims=True)
                S_ref[...] = (S_ref[...] * scale
                              + _dot_nn(wT, xt, lax.Precision.DEFAULT))
                cnt_ref[...] = cnt_ref[...] + jnp.sum(oh.astype(jnp.float32),
                                                      axis=1, keepdims=True)

            return carry

        lax.fori_loop(0, NT, tile_body, 0)

        s = s_ref[...]
        S = S_ref[...]
        cnt = cnt_ref[...]
        r = jnp.where(s > 0.0, S / jnp.where(s > 0.0, s, 1.0), 0.0)
        r = r / jnp.maximum(cnt, 1.0)

        q = q_ref[...]
        inp = jnp.concatenate([q, r], axis=1)
        gates = _dot_nt(inp, wih) + _dot_nt(h, whh) + bias
        i_g = jax.nn.sigmoid(gates[:, 0:D])
        f_g = jax.nn.sigmoid(gates[:, D:2 * D])
        g_g = jnp.tanh(gates[:, 2 * D:3 * D])
        o_g = jax.nn.sigmoid(gates[:, 3 * D:4 * D])
        c = f_g * c + i_g * g_g
        h = o_g * jnp.tanh(c)
        q_ref[...] = h

    out_ref[...] = jnp.concatenate([q_ref[...], r], axis=1)


@jax.jit
def kernel(x, batch, W_ih, W_hh, b_ih, b_hh):
    x3 = x.reshape(NT, TILE, D)
    b3 = batch.astype(jnp.int32).reshape(NT, 1, TILE)
    bias = (b_ih + b_hh).reshape(1, 4 * D)
    return pl.pallas_call(
        _set2set_kernel,
        out_shape=jax.ShapeDtypeStruct((B, 2 * D), jnp.float32),
        scratch_shapes=[
            pltpu.VMEM((B, D), jnp.float32),
            pltpu.VMEM((B, 1), jnp.float32),
            pltpu.VMEM((B, 1), jnp.float32),
            pltpu.VMEM((B, 1), jnp.float32),
            pltpu.VMEM((B, D), jnp.float32),
        ],
        compiler_params=pltpu.CompilerParams(
            vmem_limit_bytes=120 * 1024 * 1024,
        ),
    )(x3, b3, W_ih, W_hh, bias)


# speedup vs baseline: 31.8390x; 1.2254x over previous
import jax
import jax.numpy as jnp
from jax import lax
from jax.experimental import pallas as pl
from jax.experimental.pallas import tpu as pltpu

N = 100000
D = 128
B = 512
STEPS = 4
TILE = 2000
NT = N // TILE
W = 64

_NEG = -1e30


def _dot_nt(a, b, precision=lax.Precision.HIGHEST):
    return lax.dot_general(a, b, (((1,), (1,)), ((), ())),
                           precision=precision, preferred_element_type=jnp.float32)


def _dot_nn(a, b, precision=lax.Precision.HIGHEST):
    return lax.dot_general(a, b, (((1,), (0,)), ((), ())),
                           precision=precision, preferred_element_type=jnp.float32)


def _set2set_kernel(x_ref, b_ref, wih_ref, whh_ref, bias_ref, out_ref,
                    q_ref, m_ref, s_ref, cnt_ref, S_ref):
    wih = wih_ref[...]
    whh = whh_ref[...]
    bias = bias_ref[...]

    q_ref[...] = jnp.zeros((B, D), jnp.float32)
    h = jnp.zeros((B, D), jnp.float32)
    c = jnp.zeros((B, D), jnp.float32)

    iota_w = lax.broadcasted_iota(jnp.int32, (W, TILE), 0)
    iota_f = lax.broadcasted_iota(jnp.int32, (B, TILE), 0)

    r = None
    for step in range(STEPS):
        m_ref[...] = jnp.full((B, 1), _NEG, jnp.float32)
        s_ref[...] = jnp.zeros((B, 1), jnp.float32)
        S_ref[...] = jnp.zeros((B, D), jnp.float32)
        do_cnt = step == 0
        if do_cnt:
            cnt_ref[...] = jnp.zeros((B, 1), jnp.float32)

        def tile_body(t, carry):
            xt = x_ref[t]
            btr = b_ref[t]
            b0a = jnp.minimum((jnp.min(btr) // 8) * 8, B - W)
            wide = (jnp.max(btr) - b0a) >= W

            @pl.when(jnp.logical_not(wide))
            def narrow():
                oh = (iota_w + b0a) == btr
                qw = q_ref[pl.ds(b0a, W), :]
                E = _dot_nt(qw, xt, lax.Precision.DEFAULT)
                e_row = jnp.sum(jnp.where(oh, E, 0.0), axis=0, keepdims=True)
                mt = jnp.max(jnp.where(oh, E, _NEG), axis=1, keepdims=True)
                mw = m_ref[pl.ds(b0a, W), :]
                mw_new = jnp.maximum(mw, mt)
                scale = jnp.exp(mw - mw_new)
                g_row = jnp.sum(jnp.where(oh, mw_new, 0.0), axis=0, keepdims=True)
                ex = jnp.exp(e_row - g_row)
                wT = jnp.where(oh, ex, 0.0)
                m_ref[pl.ds(b0a, W), :] = mw_new
                s_ref[pl.ds(b0a, W), :] = (s_ref[pl.ds(b0a, W), :] * scale
                                           + jnp.sum(wT, axis=1, keepdims=True))
                S_ref[pl.ds(b0a, W), :] = (S_ref[pl.ds(b0a, W), :] * scale
                                           + _dot_nn(wT, xt, lax.Precision.DEFAULT))
                if do_cnt:
                    cnt_ref[pl.ds(b0a, W), :] = (cnt_ref[pl.ds(b0a, W), :]
                                                 + jnp.sum(oh.astype(jnp.float32),
                                                           axis=1, keepdims=True))

            @pl.when(wide)
            def full():
                oh = iota_f == btr
                E = _dot_nt(q_ref[...], xt, lax.Precision.DEFAULT)
                e_row = jnp.sum(jnp.where(oh, E, 0.0), axis=0, keepdims=True)
                mt = jnp.max(jnp.where(oh, E, _NEG), axis=1, keepdims=True)
                m = m_ref[...]
                m_new = jnp.maximum(m, mt)
                scale = jnp.exp(m - m_new)
                g_row = jnp.sum(jnp.where(oh, m_new, 0.0), axis=0, keepdims=True)
                ex = jnp.exp(e_row - g_row)
                wT = jnp.where(oh, ex, 0.0)
                m_ref[...] = m_new
                s_ref[...] = s_ref[...] * scale + jnp.sum(wT, axis=1, keepdims=True)
                S_ref[...] = (S_ref[...] * scale
                              + _dot_nn(wT, xt, lax.Precision.DEFAULT))
                if do_cnt:
                    cnt_ref[...] = cnt_ref[...] + jnp.sum(oh.astype(jnp.float32),
                                                          axis=1, keepdims=True)

            return carry

        lax.fori_loop(0, NT, tile_body, 0)

        s = s_ref[...]
        S = S_ref[...]
        cnt = cnt_ref[...]
        r = jnp.where(s > 0.0, S / jnp.where(s > 0.0, s, 1.0), 0.0)
        r = r / jnp.maximum(cnt, 1.0)

        q = q_ref[...]
        inp = jnp.concatenate([q, r], axis=1)
        gates = _dot_nt(inp, wih) + _dot_nt(h, whh) + bias
        i_g = jax.nn.sigmoid(gates[:, 0:D])
        f_g = jax.nn.sigmoid(gates[:, D:2 * D])
        g_g = jnp.tanh(gates[:, 2 * D:3 * D])
        o_g = jax.nn.sigmoid(gates[:, 3 * D:4 * D])
        c = f_g * c + i_g * g_g
        h = o_g * jnp.tanh(c)
        q_ref[...] = h

    out_ref[...] = jnp.concatenate([q_ref[...], r], axis=1)


@jax.jit
def kernel(x, batch, W_ih, W_hh, b_ih, b_hh):
    x3 = x.reshape(NT, TILE, D)
    b3 = batch.astype(jnp.int32).reshape(NT, 1, TILE)
    bias = (b_ih + b_hh).reshape(1, 4 * D)
    return pl.pallas_call(
        _set2set_kernel,
        out_shape=jax.ShapeDtypeStruct((B, 2 * D), jnp.float32),
        scratch_shapes=[
            pltpu.VMEM((B, D), jnp.float32),
            pltpu.VMEM((B, 1), jnp.float32),
            pltpu.VMEM((B, 1), jnp.float32),
            pltpu.VMEM((B, 1), jnp.float32),
            pltpu.VMEM((B, D), jnp.float32),
        ],
        compiler_params=pltpu.CompilerParams(
            vmem_limit_bytes=64 * 1024 * 1024,
        ),
    )(x3, b3, W_ih, W_hh, bias)


# speedup vs baseline: 34.0548x; 1.0696x over previous
import jax
import jax.numpy as jnp
from jax import lax
from jax.experimental import pallas as pl
from jax.experimental.pallas import tpu as pltpu

N = 100000
D = 128
B = 512
STEPS = 4
TILE = 2000
NT = N // TILE
W = 32

_NEG = -1e30


def _dot_nt(a, b, precision=lax.Precision.HIGHEST):
    return lax.dot_general(a, b, (((1,), (1,)), ((), ())),
                           precision=precision, preferred_element_type=jnp.float32)


def _dot_nn(a, b, precision=lax.Precision.HIGHEST):
    return lax.dot_general(a, b, (((1,), (0,)), ((), ())),
                           precision=precision, preferred_element_type=jnp.float32)


def _set2set_kernel(x_ref, b_ref, wih_ref, whh_ref, bias_ref, out_ref,
                    q_ref, m_ref, s_ref, cnt_ref, S_ref):
    wih = wih_ref[...]
    whh = whh_ref[...]
    bias = bias_ref[...]

    q_ref[...] = jnp.zeros((B, D), jnp.float32)
    h = jnp.zeros((B, D), jnp.float32)
    c = jnp.zeros((B, D), jnp.float32)

    iota_w = lax.broadcasted_iota(jnp.int32, (W, TILE), 0)
    iota_f = lax.broadcasted_iota(jnp.int32, (B, TILE), 0)

    r = None
    for step in range(STEPS):
        m_ref[...] = jnp.full((B, 1), _NEG, jnp.float32)
        s_ref[...] = jnp.zeros((B, 1), jnp.float32)
        S_ref[...] = jnp.zeros((B, D), jnp.float32)
        do_cnt = step == 0
        if do_cnt:
            cnt_ref[...] = jnp.zeros((B, 1), jnp.float32)

        def tile_body(t, carry):
            xt = x_ref[t]
            btr = b_ref[t]
            b0a = jnp.minimum((jnp.min(btr) // 8) * 8, B - W)
            wide = (jnp.max(btr) - b0a) >= W

            @pl.when(jnp.logical_not(wide))
            def narrow():
                oh = (iota_w + b0a) == btr
                qw = q_ref[pl.ds(b0a, W), :]
                E = _dot_nt(qw, xt, lax.Precision.DEFAULT)
                e_row = jnp.sum(jnp.where(oh, E, 0.0), axis=0, keepdims=True)
                mt = jnp.max(jnp.where(oh, E, _NEG), axis=1, keepdims=True)
                mw = m_ref[pl.ds(b0a, W), :]
                mw_new = jnp.maximum(mw, mt)
                scale = jnp.exp(mw - mw_new)
                g_row = jnp.sum(jnp.where(oh, mw_new, 0.0), axis=0, keepdims=True)
                ex = jnp.exp(e_row - g_row)
                wT = jnp.where(oh, ex, 0.0)
                m_ref[pl.ds(b0a, W), :] = mw_new
                s_ref[pl.ds(b0a, W), :] = (s_ref[pl.ds(b0a, W), :] * scale
                                           + jnp.sum(wT, axis=1, keepdims=True))
                S_ref[pl.ds(b0a, W), :] = (S_ref[pl.ds(b0a, W), :] * scale
                                           + _dot_nn(wT, xt, lax.Precision.DEFAULT))
                if do_cnt:
                    cnt_ref[pl.ds(b0a, W), :] = (cnt_ref[pl.ds(b0a, W), :]
                                                 + jnp.sum(oh.astype(jnp.float32),
                                                           axis=1, keepdims=True))

            @pl.when(wide)
            def full():
                oh = iota_f == btr
                E = _dot_nt(q_ref[...], xt, lax.Precision.DEFAULT)
                e_row = jnp.sum(jnp.where(oh, E, 0.0), axis=0, keepdims=True)
                mt = jnp.max(jnp.where(oh, E, _NEG), axis=1, keepdims=True)
                m = m_ref[...]
                m_new = jnp.maximum(m, mt)
                scale = jnp.exp(m - m_new)
                g_row = jnp.sum(jnp.where(oh, m_new, 0.0), axis=0, keepdims=True)
                ex = jnp.exp(e_row - g_row)
                wT = jnp.where(oh, ex, 0.0)
                m_ref[...] = m_new
                s_ref[...] = s_ref[...] * scale + jnp.sum(wT, axis=1, keepdims=True)
                S_ref[...] = (S_ref[...] * scale
                              + _dot_nn(wT, xt, lax.Precision.DEFAULT))
                if do_cnt:
                    cnt_ref[...] = cnt_ref[...] + jnp.sum(oh.astype(jnp.float32),
                                                          axis=1, keepdims=True)

            return carry

        lax.fori_loop(0, NT, tile_body, 0)

        s = s_ref[...]
        S = S_ref[...]
        cnt = cnt_ref[...]
        r = jnp.where(s > 0.0, S / jnp.where(s > 0.0, s, 1.0), 0.0)
        r = r / jnp.maximum(cnt, 1.0)

        q = q_ref[...]
        inp = jnp.concatenate([q, r], axis=1)
        gates = _dot_nt(inp, wih) + _dot_nt(h, whh) + bias
        i_g = jax.nn.sigmoid(gates[:, 0:D])
        f_g = jax.nn.sigmoid(gates[:, D:2 * D])
        g_g = jnp.tanh(gates[:, 2 * D:3 * D])
        o_g = jax.nn.sigmoid(gates[:, 3 * D:4 * D])
        c = f_g * c + i_g * g_g
        h = o_g * jnp.tanh(c)
        q_ref[...] = h

    out_ref[...] = jnp.concatenate([q_ref[...], r], axis=1)


@jax.jit
def kernel(x, batch, W_ih, W_hh, b_ih, b_hh):
    x3 = x.reshape(NT, TILE, D)
    b3 = batch.astype(jnp.int32).reshape(NT, 1, TILE)
    bias = (b_ih + b_hh).reshape(1, 4 * D)
    return pl.pallas_call(
        _set2set_kernel,
        out_shape=jax.ShapeDtypeStruct((B, 2 * D), jnp.float32),
        scratch_shapes=[
            pltpu.VMEM((B, D), jnp.float32),
            pltpu.VMEM((B, 1), jnp.float32),
            pltpu.VMEM((B, 1), jnp.float32),
            pltpu.VMEM((B, 1), jnp.float32),
            pltpu.VMEM((B, D), jnp.float32),
        ],
        compiler_params=pltpu.CompilerParams(
            vmem_limit_bytes=64 * 1024 * 1024,
        ),
    )(x3, b3, W_ih, W_hh, bias)
